# baseline (speedup 1.0000x reference)
import functools

import jax
import jax.numpy as jnp
from jax import lax
from jax.experimental import pallas as pl
from jax.experimental.pallas import tpu as pltpu
from jax.experimental.pallas import tpu_sc as plsc

_H, _V, _D = 4, 100000, 64
_VMAIN = (_V // 128) * 128
_NVT = _VMAIN // 128
_PPH = _V // 2
_R = 51200


def _pack_fn(tt, tail):
    info = plsc.get_sparse_core_info()
    NC = info.num_cores
    NW = NC * info.num_subcores
    NBLK = _H * _NVT
    per_w = 2 * (-(-NBLK // (2 * NW)))
    mesh = plsc.VectorSubcoreMesh(core_axis_name="c", subcore_axis_name="s", num_cores=NC, num_subcores=info.num_subcores)

    @functools.partial(
        pl.kernel,
        out_type=jax.ShapeDtypeStruct((_H * _PPH, 128), jnp.float32),
        mesh=mesh,
        scratch_types=[
            [pltpu.VMEM((_D, 128), jnp.float32) for _ in range(2)],
            [pltpu.VMEM((64, 128), jnp.float32) for _ in range(2)],
            [pltpu.SemaphoreType.DMA for _ in range(2)],
            [pltpu.SemaphoreType.DMA for _ in range(2)],
        ],
        compiler_params=pltpu.CompilerParams(
            use_tc_tiling_on_sc=True, needs_layout_passes=False),
    )
    def k(tt_h, tail_h, ws, blk, pr, sem_r, sem_w):
        wid = lax.axis_index("s") * NC + lax.axis_index("c")
        bid0 = wid * per_w
        rows_g = [lax.iota(jnp.int32, 16) + g * 16 for g in range(4)]

        def fetch(bid, buf):
            h = bid // _NVT
            vt = bid - h * _NVT
            for dt in range(_D // 8):
                pltpu.async_copy(
                    tt_h.at[h, pl.ds(dt * 8, 8), pl.ds(vt * 128, 128)],
                    blk[buf].at[pl.ds(dt * 8, 8)], sem_r[buf])

        def drain_fetch(buf):
            for dt in range(_D // 8):
                pltpu.make_async_copy(
                    tt_h.at[0, pl.ds(0, 8), pl.ds(0, 128)],
                    blk[buf].at[pl.ds(dt * 8, 8)], sem_r[buf]).wait()

        def drain_write(buf):
            pltpu.make_async_copy(
                pr[buf], ws.at[pl.ds(0, 64)], sem_w[buf]).wait()

        def pack(bid, buf):
            def body(j, carry):
                ce = jnp.full((16,), 2 * j, jnp.int32)
                co = ce + 1
                for g in range(4):
                    pr[buf][j, pl.ds(g * 16, 16)] = plsc.load_gather(
                        blk[buf], [rows_g[g], ce])
                    pr[buf][j, pl.ds(64 + g * 16, 16)] = plsc.load_gather(
                        blk[buf], [rows_g[g], co])
                return carry
            lax.fori_loop(0, 64, body, 0, unroll=4)
            h = bid // _NVT
            vt = bid - h * _NVT
            pltpu.async_copy(
                pr[buf], ws.at[pl.ds(h * _PPH + vt * 64, 64)], sem_w[buf])

        @pl.when(bid0 < NBLK)
        def _():
            fetch(bid0, 0)

        def body(u, carry):
            b0 = bid0 + 2 * u
            b1 = b0 + 1

            @pl.when(b1 < NBLK)
            def _():
                fetch(b1, 1)

            @pl.when(b0 < NBLK)
            def _():
                drain_fetch(0)

                @pl.when(u > 0)
                def _():
                    drain_write(0)
                pack(b0, 0)

            @pl.when((b0 + 2 < NBLK) & (u < per_w // 2 - 1))
            def _():
                fetch(b0 + 2, 0)

            @pl.when(b1 < NBLK)
            def _():
                drain_fetch(1)

                @pl.when(u > 0)
                def _():
                    drain_write(1)
                pack(b1, 1)
            return carry

        lax.fori_loop(0, per_w // 2, body, 0)

        @pl.when(bid0 < NBLK)
        def _():
            drain_write(0)

        @pl.when(bid0 + 1 < NBLK)
        def _():
            drain_write(1)

        @pl.when(wid == NW - 1)
        def _():
            for h in range(_H):
                pltpu.sync_copy(
                    tail_h.at[pl.ds(h * 16, 16)],
                    ws.at[pl.ds(h * _PPH + (_VMAIN // 2), 16)])

    return k(tt, tail)


def _gather_fn(xh, ws):
    info = plsc.get_sparse_core_info()
    NC = info.num_cores
    NW = NC * info.num_subcores
    rpw = _R // NW
    C = 80
    NCH = rpw // C
    G = C // 16
    mesh = plsc.VectorSubcoreMesh(core_axis_name="c", subcore_axis_name="s", num_cores=NC, num_subcores=info.num_subcores)

    @functools.partial(
        pl.kernel,
        out_type=jax.ShapeDtypeStruct((_R, 128), jnp.float32),
        mesh=mesh,
        scratch_types=[
            [pltpu.VMEM((rpw,), jnp.int32) for _ in range(_H)],
            [pltpu.VMEM((rpw,), jnp.int32) for _ in range(_H)],
            [[pltpu.VMEM((C, 128), jnp.float32) for _ in range(_H)]
             for _ in range(2)],
            [pltpu.VMEM((C, 128), jnp.float32) for _ in range(2)],
            pltpu.SemaphoreType.DMA,
            [pltpu.SemaphoreType.DMA for _ in range(2)],
            [pltpu.SemaphoreType.DMA for _ in range(2)],
        ],
        compiler_params=pltpu.CompilerParams(
            use_tc_tiling_on_sc=True, needs_layout_passes=False),
    )
    def k(x_hbm, ws_hbm, out_hbm, pidx, parc, gb, acc, sem_i, sem_g, sem_o):
        wid = lax.axis_index("s") * NC + lax.axis_index("c")
        base = wid * rpw
        iota = lax.iota(jnp.int32, 16)

        cps = [pltpu.async_copy(x_hbm.at[pl.ds(h * _R + base, rpw)], pidx[h],
                                sem_i) for h in range(_H)]
        for cp in cps:
            cp.wait()

        def conv(i, carry):
            for h in range(_H):
                v = pidx[h][pl.ds(i * 16, 16)]
                parc[h][pl.ds(i * 16, 16)] = (v & 1) * 64
                pidx[h][pl.ds(i * 16, 16)] = (v >> 1) + h * _PPH
            return carry
        lax.fori_loop(0, rpw // 16, conv, 0, unroll=4)

        def fire(g, buf):
            for h in range(_H):
                pltpu.async_copy(
                    ws_hbm.at[pidx[h].at[pl.ds(g * C, C)]], gb[buf][h],
                    sem_g[buf])

        def drain_gather(buf):
            for h in range(_H):
                pltpu.make_async_copy(
                    ws_hbm.at[pl.ds(0, C)], gb[buf][h], sem_g[buf]).wait()

        def drain_out(buf):
            pltpu.make_async_copy(
                acc[buf], out_hbm.at[pl.ds(0, C)], sem_o[buf]).wait()

        def select_sum(g, buf, u):
            drain_gather(buf)

            @pl.when(u > 0)
            def _():
                drain_out(buf)
            for grp in range(G):
                r16 = iota + grp * 16
                pc = [parc[h][pl.ds(g * C + grp * 16, 16)] for h in range(_H)]

                def body(d, carry):
                    v = plsc.load_gather(gb[buf][0], [r16, pc[0] + d])
                    for h in range(1, _H):
                        v = v + plsc.load_gather(gb[buf][h], [r16, pc[h] + d])
                    plsc.store_scatter(acc[buf], [r16, jnp.full((16,), d,
                                                                jnp.int32)], v)
                    return carry
                lax.fori_loop(0, _D, body, 0, unroll=8)
            pltpu.async_copy(acc[buf], out_hbm.at[pl.ds(base + g * C, C)],
                             sem_o[buf])

        fire(0, 0)

        def body(u, carry):
            g0 = 2 * u
            fire(g0 + 1, 1)
            select_sum(g0, 0, u)

            @pl.when(g0 + 2 < NCH)
            def _():
                fire(g0 + 2, 0)
            select_sum(g0 + 1, 1, u)
            return carry

        lax.fori_loop(0, NCH // 2, body, 0)
        drain_out(0)
        drain_out(1)

    return k(xh, ws)


def kernel(x, tables):
    H, V, D = tables.shape
    B, L = x.shape
    tt = tables.transpose(0, 2, 1)
    tail = tables[:, _VMAIN:, :].reshape(H, 16, 128).reshape(H * 16, 128)
    ws = _pack_fn(tt, tail)
    out2 = _gather_fn(x.reshape(B * L), ws)
    return out2[:, :D].reshape(B // H, L, D)


# speedup vs baseline: 3.1975x; 3.1975x over previous
import functools

import jax
import jax.numpy as jnp
from jax import lax
from jax.experimental import pallas as pl
from jax.experimental.pallas import tpu as pltpu
from jax.experimental.pallas import tpu_sc as plsc


def _mimo_embed_sc(xh, table_flat, H, V, D, R):
    info = plsc.get_sparse_core_info()
    NC, NS, NL = info.num_cores, info.num_subcores, info.num_lanes
    NW = NC * NS
    rpw = R // NW
    NCHUNK = 4
    C = rpw // NCHUNK
    assert rpw % NCHUNK == 0 and C % 8 == 0

    mesh = plsc.VectorSubcoreMesh(core_axis_name="c", subcore_axis_name="s")

    @functools.partial(
        pl.kernel,
        out_type=jax.ShapeDtypeStruct((R, D), jnp.float32),
        mesh=mesh,
        scratch_types=[
            [pltpu.VMEM((rpw,), jnp.int32) for _ in range(H)],
            [pltpu.VMEM((C, D), jnp.float32) for _ in range(NCHUNK)],
            pltpu.SemaphoreType.DMA,
            [pltpu.SemaphoreType.DMA for _ in range(NCHUNK)],
            [pltpu.SemaphoreType.DMA for _ in range(NCHUNK)],
            [pltpu.SemaphoreType.DMA for _ in range(NCHUNK)],
        ],
        compiler_params=pltpu.CompilerParams(use_tc_tiling_on_sc=False),
    )
    def k(x_hbm, tab_hbm, out_hbm, idx_v, acc_v, sem_i, sem_g0, sem_ga,
          sem_o):
        wid = lax.axis_index("s") * NC + lax.axis_index("c")
        base = wid * rpw

        idx_cp = [
            pltpu.async_copy(x_hbm.at[pl.ds(h * R + base, rpw)], idx_v[h],
                             sem_i)
            for h in range(H)
        ]
        for cp in idx_cp:
            cp.wait()

        def off(h):
            def body(i, carry):
                ih = idx_v[h]
                ih[pl.ds(i * NL, NL)] = ih[pl.ds(i * NL, NL)] + h * V
                return carry
            return body

        for h in range(1, H):
            lax.fori_loop(0, rpw // NL, off(h), 0)

        g0 = [
            pltpu.async_copy(
                tab_hbm.at[idx_v[0].at[pl.ds(g * C, C)]], acc_v[g],
                sem_g0[g])
            for g in range(NCHUNK)
        ]
        ga = []
        for g in range(NCHUNK):
            g0[g].wait()
            ga.append([
                pltpu.async_copy(
                    tab_hbm.at[idx_v[h].at[pl.ds(g * C, C)]], acc_v[g],
                    sem_ga[g], add=True)
                for h in range(1, H)
            ])
        ow = []
        for g in range(NCHUNK):
            for cp in ga[g]:
                cp.wait()
            ow.append(
                pltpu.async_copy(acc_v[g], out_hbm.at[pl.ds(base + g * C, C)],
                                 sem_o[g]))
        for cp in ow:
            cp.wait()

    return k(xh, table_flat)


def kernel(x, tables):
    H, V, D = tables.shape
    B, L = x.shape
    R = (B // H) * L
    xh = x.reshape(H * R)
    table_flat = tables.reshape(H * V, D)
    out = _mimo_embed_sc(xh, table_flat, H, V, D, R)
    return out.reshape(B // H, L, D)
